# baseline (speedup 1.0000x reference)
import functools

import jax
import jax.numpy as jnp
from jax import lax
from jax.experimental import pallas as pl
from jax.experimental.pallas import tpu as pltpu
from jax.experimental.pallas import tpu_sc as plsc

_NC = 2
_NS = 16
_NW = _NC * _NS
_LANES = 16

_BLK = 128
_LAG = 16


@functools.lru_cache(maxsize=None)
def _make_sc_gather(n_total, d, half_rows):
    per_w = n_total // _NW
    n_blocks = per_w // _BLK
    vecs_per_blk = _BLK // _LANES
    half = half_rows * d

    mesh = plsc.VectorSubcoreMesh(core_axis_name="c", subcore_axis_name="s")

    @functools.partial(
        pl.kernel,
        mesh=mesh,
        out_type=jax.ShapeDtypeStruct((n_total,), jnp.float32),
        scratch_types=[
            pltpu.VMEM((per_w,), jnp.int32),
            pltpu.VMEM((per_w,), jnp.float32),
            pltpu.VMEM((per_w,), jnp.float32),
            pltpu.VMEM((_LAG, _BLK), jnp.int32),
            pltpu.VMEM((_LAG, _BLK), jnp.int32),
            pltpu.SemaphoreType.DMA,
        ],
    )
    def sc_gather(lo_hbm, hi_hbm, idx_hbm, out_hbm,
                  raw_v, out_v, hiv_v, rlo_v, rhi_v, sem):
        wid = lax.axis_index("s") * _NC + lax.axis_index("c")
        base = wid * per_w

        pltpu.sync_copy(idx_hbm.at[pl.ds(base, per_w)], raw_v)

        def fire(g, b):
            for v in range(vecs_per_blk):
                off = pl.multiple_of(g * _BLK + v * _LANES, _LANES)
                col = lax.iota(jnp.int32, _LANES) + ((v * _LANES) % d)
                fid = raw_v[pl.ds(off, _LANES)] * d + col
                sub = pl.ds(v * _LANES, _LANES)
                rlo_v[b, sub] = jnp.minimum(fid, half - 1)
                rhi_v[b, sub] = jnp.maximum(fid - half, 0)
            boff = pl.multiple_of(g * _BLK, _BLK)
            pltpu.async_copy(
                lo_hbm.at[rlo_v.at[b]], out_v.at[pl.ds(boff, _BLK)], sem
            )
            pltpu.async_copy(
                hi_hbm.at[rhi_v.at[b]], hiv_v.at[pl.ds(boff, _BLK)], sem
            )

        def drain(g, b):
            boff = pl.multiple_of(g * _BLK, _BLK)
            pltpu.make_async_copy(
                lo_hbm.at[rlo_v.at[b]], out_v.at[pl.ds(boff, _BLK)], sem
            ).wait()
            pltpu.make_async_copy(
                hi_hbm.at[rhi_v.at[b]], hiv_v.at[pl.ds(boff, _BLK)], sem
            ).wait()

        def select(g):
            for v in range(vecs_per_blk):
                off = pl.multiple_of(g * _BLK + v * _LANES, _LANES)
                sub = pl.ds(off, _LANES)
                keep_lo = raw_v[sub] < half_rows
                out_v[sub] = jnp.where(keep_lo, out_v[sub], hiv_v[sub])

        for b in range(_LAG):
            fire(b, b)

        def loop_body(g2, carry):
            for b in range(_LAG):
                g = g2 * _LAG + b
                drain(g, b)
                select(g)

                @pl.when(g + _LAG < n_blocks)
                def _():
                    fire(g + _LAG, b)

            return carry

        lax.fori_loop(0, n_blocks // _LAG, loop_body, 0)

        pltpu.sync_copy(out_v, out_hbm.at[pl.ds(base, per_w)])

    return sc_gather


def kernel(data, indices, axis):
    del axis
    v, d = data.shape
    r, c = indices.shape
    n_total = r * c
    assert c == d
    assert d % _LANES == 0 and _BLK % d == 0
    assert n_total % (_NW * _BLK * _LAG) == 0
    half_rows = v // 2
    assert half_rows * 2 == v and half_rows % 8 == 0

    flat_lo = data[:half_rows].reshape(half_rows * d)
    pred = indices[0, 0] >= jnp.int32(-1)
    flat_hi = jnp.where(pred, data[half_rows:], jnp.float32(0)).reshape(
        half_rows * d
    )
    flat_idx = indices.reshape(n_total)
    out = _make_sc_gather(n_total, d, half_rows)(flat_lo, flat_hi, flat_idx)
    return out.reshape(r, c)


# speedup vs baseline: 5.0112x; 5.0112x over previous
import functools

import jax
import jax.numpy as jnp
from jax import lax
from jax.experimental import pallas as pl
from jax.experimental.pallas import tpu as pltpu
from jax.experimental.pallas import tpu_sc as plsc

_NC = 2
_NS = 16
_NW = _NC * _NS
_LANES = 16

_BLK = 128
_LAG = 16


@functools.lru_cache(maxsize=None)
def _make_sc_gather(n_total, d):
    per_w = n_total // _NW
    n_blocks = per_w // _BLK
    vecs_per_blk = _BLK // _LANES

    mesh = plsc.VectorSubcoreMesh(core_axis_name="c", subcore_axis_name="s")

    @functools.partial(
        pl.kernel,
        mesh=mesh,
        out_type=jax.ShapeDtypeStruct((n_total,), jnp.float32),
        scratch_types=[
            pltpu.VMEM((per_w,), jnp.int32),
            pltpu.VMEM((per_w,), jnp.int32),
            pltpu.VMEM((per_w,), jnp.float32),
            pltpu.SemaphoreType.DMA,
        ],
    )
    def sc_gather(data_hbm, idx_hbm, out_hbm, raw_v, fidx_v, out_v, sem):
        wid = lax.axis_index("s") * _NC + lax.axis_index("c")
        base = wid * per_w

        pltpu.sync_copy(idx_hbm.at[pl.ds(base, per_w)], raw_v)

        def fire(g):
            for v in range(vecs_per_blk):
                off = pl.multiple_of(g * _BLK + v * _LANES, _LANES)
                col = lax.iota(jnp.int32, _LANES) + ((v * _LANES) % d)
                fidx_v[pl.ds(off, _LANES)] = raw_v[pl.ds(off, _LANES)] * d + col
            boff = pl.multiple_of(g * _BLK, _BLK)
            pltpu.async_copy(
                data_hbm.at[fidx_v.at[pl.ds(boff, _BLK)]],
                out_v.at[pl.ds(boff, _BLK)],
                sem,
            )

        def drain(g):
            boff = pl.multiple_of(g * _BLK, _BLK)
            pltpu.make_async_copy(
                data_hbm.at[fidx_v.at[pl.ds(boff, _BLK)]],
                out_v.at[pl.ds(boff, _BLK)],
                sem,
            ).wait()

        def loop_body(g, carry):
            fire(g)

            @pl.when(g >= _LAG)
            def _():
                drain(g - _LAG)

            return carry

        lax.fori_loop(0, n_blocks, loop_body, 0)

        def drain_body(g, carry):
            drain(g)
            return carry

        lax.fori_loop(n_blocks - _LAG, n_blocks, drain_body, 0)

        pltpu.sync_copy(out_v, out_hbm.at[pl.ds(base, per_w)])

    return sc_gather


def kernel(data, indices, axis):
    del axis
    v, d = data.shape
    r, c = indices.shape
    n_total = r * c
    assert c == d
    assert d % _LANES == 0 and _BLK % d == 0
    assert n_total % (_NW * _BLK) == 0

    flat_data = data.reshape(v * d)
    flat_idx = indices.reshape(n_total)
    out = _make_sc_gather(n_total, d)(flat_data, flat_idx)
    return out.reshape(r, c)
